# Initial kernel scaffold; baseline (speedup 1.0000x reference)
import jax
import jax.numpy as jnp
from jax.experimental import pallas as pl


def kernel(x, edge_index, edge_weight, Wi, bi, Wm1, bm1, Wm2, bm2, W_ih, b_ih, W_hh, b_hh, Wo, bo):
    raise NotImplementedError("write your pallas kernel here")


# speedup vs baseline: 1.7045x; 1.7045x over previous
import functools

import jax
import jax.numpy as jnp
from jax import lax
from jax.experimental import pallas as pl
from jax.experimental.pallas import tpu as pltpu

N = 10000
E = 320000
T = 4
D = 128
H = 128
BN = 1000


def _proj_body(x_ref, Wi_ref, bi_ref, out_ref):
    Wi = Wi_ref[...]
    bi = bi_ref[...]
    for t in range(T):
        out_ref[t] = jnp.dot(x_ref[:, t, :], Wi,
                             preferred_element_type=jnp.float32) + bi


def _proj(x, Wi, bi):
    return pl.pallas_call(
        _proj_body,
        grid=(N // BN,),
        in_specs=[
            pl.BlockSpec((BN, T, D), lambda i: (i, 0, 0)),
            pl.BlockSpec((D, H), lambda i: (0, 0)),
            pl.BlockSpec((1, H), lambda i: (0, 0)),
        ],
        out_specs=pl.BlockSpec((T, BN, H), lambda i: (0, i, 0)),
        out_shape=jax.ShapeDtypeStruct((T, N, H), jnp.float32),
    )(x, Wi, bi.reshape(1, H))


def _dense_body(first_t, emit_out,
                aggx_ref, aggh_ref, xp_ref, hid_ref, deg_ref,
                Wm1_ref, bm1_ref, Wm2_ref, bm2_ref,
                Wih_ref, bih_ref, Whh_ref, bhh_ref,
                Wo_ref, bo_ref,
                newh_ref, out_ref):
    inv = 1.0 / (deg_ref[:, 0] + deg_ref[:, 1] + 1.0)
    inv = inv[:, None]
    xs = (aggx_ref[0, 0] + aggx_ref[1, 0] + xp_ref[0]) * inv
    m = jnp.dot(xs, Wm1_ref[:H], preferred_element_type=jnp.float32)
    if first_t:
        hid = jnp.zeros_like(xs)
    else:
        hid = hid_ref[...]
        hs = (aggh_ref[0] + aggh_ref[1] + hid) * inv
        m = m + jnp.dot(hs, Wm1_ref[H:], preferred_element_type=jnp.float32)
    m = jax.nn.relu(m + bm1_ref[...])
    m = jnp.dot(m, Wm2_ref[...], preferred_element_type=jnp.float32) + bm2_ref[...]
    gi = jnp.dot(m, Wih_ref[...], preferred_element_type=jnp.float32) + bih_ref[...]
    if first_t:
        gh = bhh_ref[...]
        gh = jnp.broadcast_to(gh, gi.shape)
    else:
        gh = jnp.dot(hid, Whh_ref[...], preferred_element_type=jnp.float32) + bhh_ref[...]
    r = jax.nn.sigmoid(gi[:, :H] + gh[:, :H])
    z = jax.nn.sigmoid(gi[:, H:2 * H] + gh[:, H:2 * H])
    n = jnp.tanh(gi[:, 2 * H:] + r * gh[:, 2 * H:])
    newh = hid + (1.0 - z) * n + z * hid
    newh_ref[...] = newh
    if emit_out:
        out_ref[...] = jax.nn.relu(
            jnp.dot(newh, Wo_ref[...], preferred_element_type=jnp.float32)
            + bo_ref[...])


def _dense_stage(t, aggx, aggh, xp, hid, degsum,
                 Wm1, bm1, Wm2, bm2, Wih, bih, Whh, bhh, Wo, bo):
    first_t = (t == 0)
    emit_out = (t == T - 1)
    full = lambda s: pl.BlockSpec(s, lambda i: tuple(0 for _ in s))
    out_shapes = [jax.ShapeDtypeStruct((N, H), jnp.float32),
                  jax.ShapeDtypeStruct((N, H), jnp.float32)]
    res = pl.pallas_call(
        functools.partial(_dense_body, first_t, emit_out),
        grid=(N // BN,),
        in_specs=[
            pl.BlockSpec((2, 1, BN, H), lambda i: (0, t, i, 0)),
            pl.BlockSpec((2, BN, H), lambda i: (0, i, 0)),
            pl.BlockSpec((1, BN, H), lambda i: (t, i, 0)),
            pl.BlockSpec((BN, H), lambda i: (i, 0)),
            pl.BlockSpec((BN, 2), lambda i: (i, 0)),
            full((2 * H, H)), full((1, H)),
            full((H, H)), full((1, H)),
            full((H, 3 * H)), full((1, 3 * H)),
            full((H, 3 * H)), full((1, 3 * H)),
            full((H, H)), full((1, H)),
        ],
        out_specs=[pl.BlockSpec((BN, H), lambda i: (i, 0)),
                   pl.BlockSpec((BN, H), lambda i: (i, 0))],
        out_shape=out_shapes,
    )(aggx, aggh, xp, hid, degsum,
      Wm1, bm1.reshape(1, H), Wm2, bm2.reshape(1, H),
      Wih, bih.reshape(1, 3 * H), Whh, bhh.reshape(1, 3 * H),
      Wo, bo.reshape(1, H))
    return res


def kernel(x, edge_index, edge_weight, Wi, bi, Wm1, bm1, Wm2, bm2,
           W_ih, b_ih, W_hh, b_hh, Wo, bo):
    src = edge_index[0].astype(jnp.int32)
    dst = edge_index[1].astype(jnp.int32)
    w = edge_weight.astype(jnp.float32)

    xp = _proj(x, Wi, bi)

    deg = jnp.zeros((N,), jnp.float32).at[dst].add(w)
    degsum = jnp.stack([deg, jnp.zeros_like(deg)], axis=1)

    def agg(v):
        return jnp.zeros((N, H), jnp.float32).at[dst].add(v[src] * w[:, None])

    aggx = jnp.stack([jnp.stack([agg(xp[t]) for t in range(T)]),
                      jnp.zeros((T, N, H), jnp.float32)])

    hid = jnp.zeros((N, H), jnp.float32)
    zero_aggh = jnp.zeros((2, N, H), jnp.float32)
    out = None
    for t in range(T):
        if t == 0:
            aggh = zero_aggh
        else:
            a = agg(hid)
            aggh = jnp.stack([a, jnp.zeros_like(a)])
        hid, out = _dense_stage(t, aggx, aggh, xp, hid, degsum,
                                Wm1, bm1, Wm2, bm2, W_ih, b_ih,
                                W_hh, b_hh, Wo, bo)
    return out


# speedup vs baseline: 4.4061x; 2.5850x over previous
import functools

import jax
import jax.numpy as jnp
from jax import lax
from jax.experimental import pallas as pl
from jax.experimental.pallas import tpu as pltpu
from jax.experimental.pallas import tpu_sc as plsc

N = 10000
E = 320000
T = 4
D = 128
H = 128
BN = 1000

NC, NS = 2, 16
NW = NC * NS
EC = 10240
EP = NW * EC
KB = 128
NB = EC // KB
CB = 16
NCH = NB // CB
ZR = 64
NP = 10240
NP2 = NP

_mesh = plsc.VectorSubcoreMesh(core_axis_name="c", subcore_axis_name="s",
                               num_cores=NC, num_subcores=NS)

_GDN = lax.GatherDimensionNumbers(offset_dims=(), collapsed_slice_dims=(0,),
                                  start_index_map=(0,))


def _lane_splat(v16, r):
    idx = jnp.full((16, 1), r, jnp.int32)
    return lax.gather(v16, idx, _GDN, (1,),
                      mode=lax.GatherScatterMode.PROMISE_IN_BOUNDS)


def _make_sc_pass(nt, with_deg):
    out_type = [jax.ShapeDtypeStruct((NC, nt, NP, H), jnp.float32)]
    if with_deg:
        out_type.append(jax.ShapeDtypeStruct((NC, 1, NP2), jnp.float32))
    scratch = [
        pltpu.VMEM((CB, KB), jnp.int32),
        pltpu.VMEM((CB, KB), jnp.int32),
        pltpu.VMEM((CB, KB), jnp.float32),
        pltpu.VMEM((KB,), jnp.int32),
        pltpu.VMEM((KB, H), jnp.float32),
        pltpu.VMEM((ZR, H), jnp.float32),
        pltpu.VMEM_SHARED((NP, H), jnp.float32),
        pltpu.SemaphoreType.DMA,
    ]
    if with_deg:
        scratch += [
            pltpu.VMEM((640,), jnp.float32),
            pltpu.VMEM_SHARED((NP2,), jnp.float32),
        ]

    def body(vals, src_h, dst_h, w_h, *rest):
        if with_deg:
            agg_out, deg_out, idxs, idxd, wv, idxt, rows, zrows, acc, sem, zvec, dacc = rest
        else:
            agg_out, idxs, idxd, wv, idxt, rows, zrows, acc, sem = rest
        cid = lax.axis_index("c")
        sid = lax.axis_index("s")
        wid = sid * NC + cid

        zero16 = jnp.zeros((16,), jnp.float32)

        def zr_body(r, carry):
            for j in range(8):
                zrows[r, pl.ds(j * 16, 16)] = zero16
            return carry
        lax.fori_loop(0, ZR, zr_body, 0)

        if with_deg:
            for i in range(40):
                zvec[pl.ds(i * 16, 16)] = zero16
            pltpu.sync_copy(zvec, dacc.at[pl.ds(sid * 640, 640)])

        def t_body(t, carry):
            for k in range(640 // ZR):
                pltpu.sync_copy(zrows, acc.at[pl.ds(sid * 640 + k * ZR, ZR)])
            plsc.subcore_barrier()

            def c_body(c, inner):
                pltpu.sync_copy(src_h.at[wid, pl.ds(c * CB, CB)], idxs)
                pltpu.sync_copy(dst_h.at[wid, pl.ds(c * CB, CB)], idxd)
                pltpu.sync_copy(w_h.at[wid, pl.ds(c * CB, CB)], wv)

                def b_body(b, inner2):
                    for g in range(KB // 16):
                        idxt[pl.ds(g * 16, 16)] = (
                            idxs[b, pl.ds(g * 16, 16)] + t * N)
                    pltpu.async_copy(vals.at[idxt], rows, sem).wait()
                    for g in range(KB // 16):
                        w16 = wv[b, pl.ds(g * 16, 16)]
                        for r in range(16):
                            e = g * 16 + r
                            wspl = _lane_splat(w16, r)
                            for j in range(8):
                                rows[e, pl.ds(j * 16, 16)] = (
                                    rows[e, pl.ds(j * 16, 16)] * wspl)
                    pltpu.sync_copy(rows, acc.at[idxd.at[b]], add=True)
                    if with_deg:
                        @pl.when(t == 0)
                        def _():
                            pltpu.sync_copy(wv.at[b], dacc.at[idxd.at[b]],
                                            add=True)
                    return inner2
                lax.fori_loop(0, CB, b_body, inner)
                return inner
            lax.fori_loop(0, NCH, c_body, 0)
            plsc.subcore_barrier()
            pltpu.sync_copy(acc.at[pl.ds(sid * 640, 640)],
                            agg_out.at[cid, t, pl.ds(sid * 640, 640)])
            plsc.subcore_barrier()
            return carry
        lax.fori_loop(0, nt, t_body, 0)

        if with_deg:
            pltpu.sync_copy(dacc.at[pl.ds(sid * 640, 640)],
                            deg_out.at[cid, 0, pl.ds(sid * 640, 640)])

    return pl.kernel(body, out_type=tuple(out_type), mesh=_mesh,
                     scratch_types=tuple(scratch))


_xpass = _make_sc_pass(T, True)
_hpass = _make_sc_pass(1, False)


def _proj_body(x_ref, Wi_ref, bi_ref, out_ref):
    Wi = Wi_ref[...]
    bi = bi_ref[...]
    for t in range(T):
        out_ref[t] = jnp.dot(x_ref[:, t, :], Wi,
                             preferred_element_type=jnp.float32) + bi


def _proj(x, Wi, bi):
    return pl.pallas_call(
        _proj_body,
        grid=(N // BN,),
        in_specs=[
            pl.BlockSpec((BN, T, D), lambda i: (i, 0, 0)),
            pl.BlockSpec((D, H), lambda i: (0, 0)),
            pl.BlockSpec((1, H), lambda i: (0, 0)),
        ],
        out_specs=pl.BlockSpec((T, BN, H), lambda i: (0, i, 0)),
        out_shape=jax.ShapeDtypeStruct((T, N, H), jnp.float32),
    )(x, Wi, bi.reshape(1, H))


def _dense_body(first_t, emit_out,
                aggx_ref, aggh_ref, xp_ref, hid_ref, deg_ref,
                Wm1_ref, bm1_ref, Wm2_ref, bm2_ref,
                Wih_ref, bih_ref, Whh_ref, bhh_ref,
                Wo_ref, bo_ref,
                newh_ref, out_ref):
    inv = 1.0 / (deg_ref[:, 0] + deg_ref[:, 1] + 1.0)
    inv = inv[:, None]
    xs = (aggx_ref[0, 0] + aggx_ref[1, 0] + xp_ref[0]) * inv
    m = jnp.dot(xs, Wm1_ref[:H], preferred_element_type=jnp.float32)
    if first_t:
        hid = jnp.zeros_like(xs)
    else:
        hid = hid_ref[...]
        hs = (aggh_ref[0, 0] + aggh_ref[1, 0] + hid) * inv
        m = m + jnp.dot(hs, Wm1_ref[H:], preferred_element_type=jnp.float32)
    m = jax.nn.relu(m + bm1_ref[...])
    m = jnp.dot(m, Wm2_ref[...], preferred_element_type=jnp.float32) + bm2_ref[...]
    gi = jnp.dot(m, Wih_ref[...], preferred_element_type=jnp.float32) + bih_ref[...]
    if first_t:
        gh = jnp.broadcast_to(bhh_ref[...], gi.shape)
    else:
        gh = jnp.dot(hid, Whh_ref[...], preferred_element_type=jnp.float32) + bhh_ref[...]
    r = jax.nn.sigmoid(gi[:, :H] + gh[:, :H])
    z = jax.nn.sigmoid(gi[:, H:2 * H] + gh[:, H:2 * H])
    n = jnp.tanh(gi[:, 2 * H:] + r * gh[:, 2 * H:])
    newh = hid + (1.0 - z) * n + z * hid
    newh_ref[...] = newh
    if emit_out:
        out_ref[...] = jax.nn.relu(
            jnp.dot(newh, Wo_ref[...], preferred_element_type=jnp.float32)
            + bo_ref[...])


def _dense_stage(t, aggx, aggh, xp, hid, degsum,
                 Wm1, bm1, Wm2, bm2, Wih, bih, Whh, bhh, Wo, bo):
    first_t = (t == 0)
    emit_out = (t == T - 1)
    full = lambda s: pl.BlockSpec(s, lambda i: tuple(0 for _ in s))
    out_shapes = [jax.ShapeDtypeStruct((N, H), jnp.float32),
                  jax.ShapeDtypeStruct((N, H), jnp.float32)]
    return pl.pallas_call(
        functools.partial(_dense_body, first_t, emit_out),
        grid=(N // BN,),
        in_specs=[
            pl.BlockSpec((2, 1, BN, H), lambda i: (0, t, i, 0)),
            pl.BlockSpec((2, 1, BN, H), lambda i: (0, 0, i, 0)),
            pl.BlockSpec((1, BN, H), lambda i: (t, i, 0)),
            pl.BlockSpec((BN, H), lambda i: (i, 0)),
            pl.BlockSpec((BN, 2), lambda i: (i, 0)),
            full((2 * H, H)), full((1, H)),
            full((H, H)), full((1, H)),
            full((H, 3 * H)), full((1, 3 * H)),
            full((H, 3 * H)), full((1, 3 * H)),
            full((H, H)), full((1, H)),
        ],
        out_specs=[pl.BlockSpec((BN, H), lambda i: (i, 0)),
                   pl.BlockSpec((BN, H), lambda i: (i, 0))],
        out_shape=out_shapes,
    )(aggx, aggh, xp, hid, degsum,
      Wm1, bm1.reshape(1, H), Wm2, bm2.reshape(1, H),
      Wih, bih.reshape(1, 3 * H), Whh, bhh.reshape(1, 3 * H),
      Wo, bo.reshape(1, H))


def kernel(x, edge_index, edge_weight, Wi, bi, Wm1, bm1, Wm2, bm2,
           W_ih, b_ih, W_hh, b_hh, Wo, bo):
    src = edge_index[0].astype(jnp.int32)
    dst = edge_index[1].astype(jnp.int32)
    w = edge_weight.astype(jnp.float32)

    pad = EP - E
    src2 = jnp.concatenate([src, jnp.zeros((pad,), jnp.int32)]).reshape(NW, NB, KB)
    dst2 = jnp.concatenate([dst, jnp.zeros((pad,), jnp.int32)]).reshape(NW, NB, KB)
    w2 = jnp.concatenate([w, jnp.zeros((pad,), jnp.float32)]).reshape(NW, NB, KB)

    xp = _proj(x, Wi, bi)

    aggx, degp = _xpass(xp.reshape(T * N, H), src2, dst2, w2)
    degsum = degp[:, 0, :N].T

    hid = jnp.zeros((N, H), jnp.float32)
    zero_aggh = jnp.zeros((NC, 1, NP, H), jnp.float32)
    out = None
    for t in range(T):
        if t == 0:
            aggh = zero_aggh
        else:
            (aggh,) = _hpass(hid, src2, dst2, w2)
        hid, out = _dense_stage(t, aggx, aggh, xp, hid, degsum,
                                Wm1, bm1, Wm2, bm2, W_ih, b_ih,
                                W_hh, b_hh, Wo, bo)
    return out
